# baseline (speedup 1.0000x reference)
import functools

import jax
import jax.numpy as jnp
from jax import lax
from jax.experimental import pallas as pl
from jax.experimental.pallas import tpu as pltpu
from jax.experimental.pallas import tpu_sc as plsc

B = 16384
D = 64
NC = 2
NS = 16
L = 16
NW = NC * NS
BPW = B // NW
NGROUP = BPW // L


def _rsqrt(x):
    i = plsc.bitcast(x, jnp.int32)
    i = jnp.int32(0x5F3759DF) - (i >> 1)
    y = plsc.bitcast(i, jnp.float32)
    for _ in range(3):
        y = y * (jnp.float32(1.5) - jnp.float32(0.5) * x * y * y)
    return y


_mesh = plsc.VectorSubcoreMesh(core_axis_name="c", subcore_axis_name="s")


@functools.partial(
    pl.kernel,
    mesh=_mesh,
    out_type=jax.ShapeDtypeStruct((B,), jnp.float32),
    scratch_types=[
        pltpu.VMEM((BPW,), jnp.int32),
        pltpu.VMEM((BPW,), jnp.int32),
        pltpu.VMEM((BPW,), jnp.int32),
        pltpu.VMEM((BPW,), jnp.int32),
        pltpu.VMEM((L, 8, D), jnp.float32),
        pltpu.VMEM((L, 8, D), jnp.float32),
        pltpu.VMEM((L, 8, D), jnp.float32),
        pltpu.VMEM((L, 8, D), jnp.float32),
        pltpu.VMEM((BPW,), jnp.float32),
        pltpu.SemaphoreType.DMA,
        pltpu.SemaphoreType.DMA,
        pltpu.SemaphoreType.DMA,
        pltpu.SemaphoreType.DMA,
    ],
    compiler_params=pltpu.CompilerParams(needs_layout_passes=False),
)
def _sc_kernel(
    users, items, utab, itab, out,
    uidx, iidx, ublk, iblk, ub0, ub1, vb0, vb1, outv, us0, us1, vs0, vs1,
):
    wid = lax.axis_index("s") * NC + lax.axis_index("c")
    base = wid * BPW

    pltpu.sync_copy(users.at[pl.ds(base, BPW)], uidx)
    pltpu.sync_copy(items.at[pl.ds(base, BPW)], iidx)

    def split_body(jv, carry):
        sl = pl.ds(jv * L, L)
        ublk[sl] = uidx[sl] >> 3
        iblk[sl] = iidx[sl] >> 3
        return carry

    lax.fori_loop(0, BPW // L, split_body, 0)

    lane = lax.iota(jnp.int32, L)

    def fire(gi, ub, vb, us, vs):
        uvec = ublk[pl.ds(gi * L, L)]
        ivec = iblk[pl.ds(gi * L, L)]
        for b in range(L):
            pltpu.async_copy(utab.at[pl.ds(uvec[b], 1)], ub.at[pl.ds(b, 1)], us)
            pltpu.async_copy(itab.at[pl.ds(ivec[b], 1)], vb.at[pl.ds(b, 1)], vs)

    def drain(ub, vb, us, vs):
        pltpu.make_async_copy(utab.at[pl.ds(0, L)], ub, us).wait()
        pltpu.make_async_copy(itab.at[pl.ds(0, L)], vb, vs).wait()

    def compute(gi, ub, vb):
        urow = uidx[pl.ds(gi * L, L)] & 7
        vrow = iidx[pl.ds(gi * L, L)] & 7
        dot = jnp.zeros((L,), jnp.float32)
        uu = jnp.zeros((L,), jnp.float32)
        vv = jnp.zeros((L,), jnp.float32)
        for c in range(D):
            cols = jnp.full((L,), c, jnp.int32)
            u = plsc.load_gather(ub, [lane, urow, cols])
            v = plsc.load_gather(vb, [lane, vrow, cols])
            dot = dot + u * v
            uu = uu + u * u
            vv = vv + v * v
        denom2 = jnp.maximum(uu, jnp.float32(1e-16)) * jnp.maximum(
            vv, jnp.float32(1e-16)
        )
        cos = dot * _rsqrt(denom2)
        outv[pl.ds(gi * L, L)] = cos * jnp.float32(2.25) + jnp.float32(2.75)

    fire(0, ub0, vb0, us0, vs0)

    def pair_body(go, carry):
        g0 = go * 2
        g1 = g0 + 1
        fire(g1, ub1, vb1, us1, vs1)
        drain(ub0, vb0, us0, vs0)
        compute(g0, ub0, vb0)

        @pl.when(g0 + 2 < NGROUP)
        def _():
            fire(g0 + 2, ub0, vb0, us0, vs0)

        drain(ub1, vb1, us1, vs1)
        compute(g1, ub1, vb1)
        return carry

    lax.fori_loop(0, NGROUP // 2, pair_body, 0)

    pltpu.sync_copy(outv, out.at[pl.ds(base, BPW)])


def kernel(users, items, user_table, item_table):
    nut = user_table.shape[0] // 8
    nit = item_table.shape[0] // 8
    return _sc_kernel(
        users.astype(jnp.int32),
        items.astype(jnp.int32),
        user_table.reshape(nut, 8, D),
        item_table.reshape(nit, 8, D),
    )


# speedup vs baseline: 1.0104x; 1.0104x over previous
import functools

import jax
import jax.numpy as jnp
from jax import lax
from jax.experimental import pallas as pl
from jax.experimental.pallas import tpu as pltpu
from jax.experimental.pallas import tpu_sc as plsc

B = 16384
D = 64
NC = 2
NS = 16
L = 16
NW = NC * NS
BPW = B // NW
NGROUP = BPW // L


def _rsqrt(x):
    i = plsc.bitcast(x, jnp.int32)
    i = jnp.int32(0x5F3759DF) - (i >> 1)
    y = plsc.bitcast(i, jnp.float32)
    for _ in range(3):
        y = y * (jnp.float32(1.5) - jnp.float32(0.5) * x * y * y)
    return y


_mesh = plsc.VectorSubcoreMesh(core_axis_name="c", subcore_axis_name="s")


@functools.partial(
    pl.kernel,
    mesh=_mesh,
    out_type=jax.ShapeDtypeStruct((B,), jnp.float32),
    scratch_types=[
        pltpu.VMEM((BPW,), jnp.int32),
        pltpu.VMEM((BPW,), jnp.int32),
        pltpu.VMEM((BPW,), jnp.int32),
        pltpu.VMEM((BPW,), jnp.int32),
        pltpu.VMEM((L, 8, D), jnp.float32),
        pltpu.VMEM((L, 8, D), jnp.float32),
        pltpu.VMEM((L, 8, D), jnp.float32),
        pltpu.VMEM((L, 8, D), jnp.float32),
        pltpu.VMEM((BPW,), jnp.float32),
        pltpu.SemaphoreType.DMA,
        pltpu.SemaphoreType.DMA,
        pltpu.SemaphoreType.DMA,
        pltpu.SemaphoreType.DMA,
    ],
    compiler_params=pltpu.CompilerParams(needs_layout_passes=False),
)
def _sc_kernel(
    users, items, utab, itab, out,
    uidx, iidx, ublk, iblk, ub0, ub1, vb0, vb1, outv, us0, us1, vs0, vs1,
):
    wid = lax.axis_index("s") * NC + lax.axis_index("c")
    base = wid * BPW

    pltpu.sync_copy(users.at[pl.ds(base, BPW)], uidx)
    pltpu.sync_copy(items.at[pl.ds(base, BPW)], iidx)

    def split_body(jv, carry):
        sl = pl.ds(jv * L, L)
        ublk[sl] = uidx[sl] >> 3
        iblk[sl] = iidx[sl] >> 3
        return carry

    lax.fori_loop(0, BPW // L, split_body, 0)

    lane = lax.iota(jnp.int32, L)

    def fire(gi, ub, vb, us, vs):
        uvec = ublk[pl.ds(gi * L, L)]
        ivec = iblk[pl.ds(gi * L, L)]
        for b in range(L):
            pltpu.async_copy(utab.at[pl.ds(uvec[b], 1)], ub.at[pl.ds(b, 1)], us)
            pltpu.async_copy(itab.at[pl.ds(ivec[b], 1)], vb.at[pl.ds(b, 1)], vs)

    def drain(ub, vb, us, vs):
        pltpu.make_async_copy(utab.at[pl.ds(0, L)], ub, us).wait()
        pltpu.make_async_copy(itab.at[pl.ds(0, L)], vb, vs).wait()

    def compute(gi, ub, vb):
        urow = uidx[pl.ds(gi * L, L)] & 7
        vrow = iidx[pl.ds(gi * L, L)] & 7
        dot = jnp.zeros((L,), jnp.float32)
        uu = jnp.zeros((L,), jnp.float32)
        vv = jnp.zeros((L,), jnp.float32)
        for c in range(D):
            cols = (lane + c) & (D - 1)
            u = plsc.load_gather(ub, [lane, urow, cols])
            v = plsc.load_gather(vb, [lane, vrow, cols])
            dot = dot + u * v
            uu = uu + u * u
            vv = vv + v * v
        denom2 = jnp.maximum(uu, jnp.float32(1e-16)) * jnp.maximum(
            vv, jnp.float32(1e-16)
        )
        cos = dot * _rsqrt(denom2)
        outv[pl.ds(gi * L, L)] = cos * jnp.float32(2.25) + jnp.float32(2.75)

    fire(0, ub0, vb0, us0, vs0)

    def pair_body(go, carry):
        g0 = go * 2
        g1 = g0 + 1
        fire(g1, ub1, vb1, us1, vs1)
        drain(ub0, vb0, us0, vs0)
        compute(g0, ub0, vb0)

        @pl.when(g0 + 2 < NGROUP)
        def _():
            fire(g0 + 2, ub0, vb0, us0, vs0)

        drain(ub1, vb1, us1, vs1)
        compute(g1, ub1, vb1)
        return carry

    lax.fori_loop(0, NGROUP // 2, pair_body, 0)

    pltpu.sync_copy(outv, out.at[pl.ds(base, BPW)])


def kernel(users, items, user_table, item_table):
    nut = user_table.shape[0] // 8
    nit = item_table.shape[0] // 8
    return _sc_kernel(
        users.astype(jnp.int32),
        items.astype(jnp.int32),
        user_table.reshape(nut, 8, D),
        item_table.reshape(nit, 8, D),
    )
